# baseline (speedup 1.0000x reference)
import functools
import jax
import jax.numpy as jnp
from jax import lax
from jax.experimental import pallas as pl
from jax.experimental.pallas import tpu as pltpu
from jax.experimental.pallas import tpu_sc as plsc

_K = 64
_B = 2048
_N = 16384
_NW = 32
_SPLIT = 1536
_ROWS_PER_W = (_B - _SPLIT) // _NW
_CHUNKS = _N // 16
_TC_ROWS_PER_BLOCK = 32


def _splat(s, dtype=jnp.int32):
    return jnp.full((16,), s, dtype)


def _scalar(v):
    return v[0]


_GATHER_DNUMS = lax.GatherDimensionNumbers(
    offset_dims=(), collapsed_slice_dims=(0,), start_index_map=(0,))


def _take0(vec, idx_splat):
    g = lax.gather(vec, idx_splat[:, None], _GATHER_DNUMS, slice_sizes=(1,),
                   mode=lax.GatherScatterMode.PROMISE_IN_BOUNDS)
    return g[0]


def _scan_hist(hist_ref, target, exclude_zero, lane_iota):
    accs = [hist_ref[pl.ds(16 * j, 16)] for j in range(16)]
    rcs = [lax.rev(a, (0,)) for a in accs]
    css = [plsc.cumsum(rc) for rc in rcs]

    tots = [cs[15] for cs in css]
    carries = [jnp.int32(0)] * 16
    carry = jnp.int32(0)
    for j in range(15, -1, -1):
        carries[j] = carry
        carry = carry + tots[j]

    target_s = _splat(target)
    found = jnp.bool_(False)
    B = jnp.int32(0)
    above = jnp.int32(0)
    for j in range(15, -1, -1):
        cs = css[j] + _splat(carries[j])
        hit = cs >= target_s
        if exclude_zero and j == 0:
            hit = hit & (lane_iota != 15)
        anyhit = _scalar(plsc.all_reduce_population_count(hit)) > 0
        istar = plsc.all_reduce_ffs(hit)
        cs_at = _take0(cs, istar)
        rc_at = _take0(rcs[j], istar)
        newly = anyhit & jnp.logical_not(found)
        B = jnp.where(newly, 16 * j + 15 - _scalar(istar), B)
        above = jnp.where(newly, cs_at - rc_at, above)
        found = found | anyhit
    return found, B, target - above


def _sc_body(x_hbm, o_hbm, row_v, hist_v, cb1_v, cb2_v):
    wid = lax.axis_index("s") * 2 + lax.axis_index("c")
    lane_iota = lax.iota(jnp.int32, 16)
    ones16 = jnp.ones((16,), jnp.int32)
    zeros16 = jnp.zeros((16,), jnp.int32)

    def reset_hist():
        for j in range(16):
            hist_v[pl.ds(16 * j, 16)] = zeros16

    def do_row(rr, _):
        orow = wid * _ROWS_PER_W + rr
        row = _SPLIT + orow
        pltpu.sync_copy(x_hbm.at[row], row_v)

        reset_hist()

        def passA(i, _):
            v = row_v[pl.ds(i * 16, 16)]
            bits = plsc.bitcast(jnp.maximum(v, 0.0), jnp.int32)
            e = lax.shift_right_logical(bits, 23)
            plsc.addupdate_scatter(hist_v, [e], ones16)
            return 0
        lax.fori_loop(0, _CHUNKS, passA, 0, unroll=8)

        found1, B1, r1 = _scan_hist(hist_v, jnp.int32(_K), True, lane_iota)
        B1 = jnp.where(found1, B1, -1)

        reset_hist()
        B1s = _splat(B1)

        def passB(i, w):
            v = row_v[pl.ds(i * 16, 16)]
            bits = plsc.bitcast(jnp.maximum(v, 0.0), jnp.int32)
            e = lax.shift_right_logical(bits, 23)
            match = e == B1s
            b2 = lax.shift_right_logical(bits, 15) & 0xFF
            plsc.addupdate_scatter(hist_v, [b2], ones16,
                                   mask=match)
            plsc.store_compressed(cb1_v.at[pl.ds(w, 16)], bits, mask=match)
            return w + _scalar(plsc.all_reduce_population_count(match))
        m1 = lax.fori_loop(0, _CHUNKS, passB, jnp.int32(0), unroll=8)

        found2, B2, r2 = _scan_hist(hist_v, r1, False, lane_iota)

        reset_hist()
        B2s = _splat(B2)
        m1s = _splat(m1)
        trip1 = lax.shift_right_logical(m1 + 15, 4)

        def passC(i, w):
            bits = cb1_v[pl.ds(i * 16, 16)]
            valid = (i * 16 + lane_iota) < m1s
            b2 = lax.shift_right_logical(bits, 15) & 0xFF
            match = valid & (b2 == B2s)
            b3 = lax.shift_right_logical(bits, 7) & 0xFF
            plsc.addupdate_scatter(hist_v, [b3], ones16,
                                   mask=match)
            plsc.store_compressed(cb2_v.at[pl.ds(w, 16)], bits, mask=match)
            return w + _scalar(plsc.all_reduce_population_count(match))
        m2 = lax.fori_loop(0, trip1, passC, jnp.int32(0))

        found3, B3, r3 = _scan_hist(hist_v, r2, False, lane_iota)

        reset_hist()
        B3s = _splat(B3)
        m2s = _splat(m2)
        trip2 = lax.shift_right_logical(m2 + 15, 4)

        def passD(i, _):
            bits = cb2_v[pl.ds(i * 16, 16)]
            valid = (i * 16 + lane_iota) < m2s
            b3 = lax.shift_right_logical(bits, 7) & 0xFF
            match = valid & (b3 == B3s)
            b4 = bits & 0x7F
            plsc.addupdate_scatter(hist_v, [b4], ones16,
                                   mask=match)
            return 0
        lax.fori_loop(0, trip2, passD, 0)

        found4, B4, _r4 = _scan_hist(hist_v, r3, False, lane_iota)

        t_bits = (B1 << 23) | (B2 << 15) | (B3 << 7) | B4
        t_bits = jnp.where(found1, t_bits, jnp.int32(1))
        tf = plsc.bitcast(_splat(t_bits), jnp.float32)

        def passE(i, _):
            v = row_v[pl.ds(i * 16, 16)]
            row_v[pl.ds(i * 16, 16)] = jnp.where(v >= tf, v, 0.0)
            return 0
        lax.fori_loop(0, _CHUNKS, passE, 0, unroll=8)

        pltpu.sync_copy(row_v, o_hbm.at[orow])
        return 0

    lax.fori_loop(0, _ROWS_PER_W, do_row, 0)


def _tc_block(x_ref, o_ref):
    x = x_ref[...]
    rows = x.shape[0]

    def step(bit, t_bits, cur_cnt):
        cand = t_bits | (jnp.int32(1) << bit)
        cand_f = jax.lax.bitcast_convert_type(cand, jnp.float32)
        cnt = jnp.sum((x >= cand_f).astype(jnp.int32), axis=1, keepdims=True)
        take = cnt >= _K
        return (jnp.where(take, cand, t_bits), jnp.where(take, cnt, cur_cnt))

    def head(i, carry):
        t_bits, cur_cnt = carry
        return step(30 - i, t_bits, cur_cnt)

    t0 = jnp.zeros((rows, 1), jnp.int32)
    c0 = jnp.full((rows, 1), _N, jnp.int32)
    t_bits, cur_cnt = jax.lax.fori_loop(0, 20, head, (t0, c0), unroll=True)

    def cond(carry):
        bit, _, cur_cnt = carry
        return (bit >= 0) & jnp.any(cur_cnt != _K)

    def body(carry):
        bit, t_bits, cur_cnt = carry
        t_bits, cur_cnt = step(bit, t_bits, cur_cnt)
        return (bit - 1, t_bits, cur_cnt)

    _, t_bits, _ = jax.lax.while_loop(cond, body, (jnp.int32(10), t_bits, cur_cnt))
    t_f = jax.lax.bitcast_convert_type(t_bits, jnp.float32)
    keep = (x >= t_f) & (x > 0.0)
    o_ref[...] = jnp.where(keep, x, 0.0)


@jax.jit
def kernel(x):
    sc_out = pl.kernel(
        _sc_body,
        out_type=jax.ShapeDtypeStruct((_B - _SPLIT, _N), jnp.float32),
        mesh=plsc.VectorSubcoreMesh(core_axis_name="c", subcore_axis_name="s"),
        compiler_params=pltpu.CompilerParams(needs_layout_passes=False),
        scratch_types=[
            pltpu.VMEM((_N,), jnp.float32),
            pltpu.VMEM((256,), jnp.int32),
            pltpu.VMEM((_N + 16,), jnp.int32),
            pltpu.VMEM((_N + 16,), jnp.int32),
        ],
    )(x)
    tc_full = pl.pallas_call(
        _tc_block,
        grid=(_SPLIT // _TC_ROWS_PER_BLOCK,),
        in_specs=[pl.BlockSpec((_TC_ROWS_PER_BLOCK, _N), lambda i: (i, 0))],
        out_specs=pl.BlockSpec((_TC_ROWS_PER_BLOCK, _N), lambda i: (i, 0)),
        out_shape=jax.ShapeDtypeStruct((_B, _N), x.dtype),
    )(x)
    return jax.lax.dynamic_update_slice(tc_full, sc_out, (_SPLIT, 0))


# speedup vs baseline: 1.0026x; 1.0026x over previous
import functools
import jax
import jax.numpy as jnp
from jax import lax
from jax.experimental import pallas as pl
from jax.experimental.pallas import tpu as pltpu
from jax.experimental.pallas import tpu_sc as plsc

_K = 64
_B = 2048
_N = 16384
_NW = 32
_SPLIT = 1536
_ROWS_PER_W = (_B - _SPLIT) // _NW
_CHUNKS = _N // 16
_TC_ROWS_PER_BLOCK = 32


def _splat(s, dtype=jnp.int32):
    return jnp.full((16,), s, dtype)


def _scalar(v):
    return v[0]


_GATHER_DNUMS = lax.GatherDimensionNumbers(
    offset_dims=(), collapsed_slice_dims=(0,), start_index_map=(0,))


def _take0(vec, idx_splat):
    g = lax.gather(vec, idx_splat[:, None], _GATHER_DNUMS, slice_sizes=(1,),
                   mode=lax.GatherScatterMode.PROMISE_IN_BOUNDS)
    return g[0]


def _scan_hist(hist_ref, target, exclude_zero, lane_iota):
    accs = [hist_ref[pl.ds(16 * j, 16)] for j in range(16)]
    rcs = [lax.rev(a, (0,)) for a in accs]
    css = [plsc.cumsum(rc) for rc in rcs]

    tots = [cs[15] for cs in css]
    carries = [jnp.int32(0)] * 16
    carry = jnp.int32(0)
    for j in range(15, -1, -1):
        carries[j] = carry
        carry = carry + tots[j]

    target_s = _splat(target)
    found = jnp.bool_(False)
    B = jnp.int32(0)
    above = jnp.int32(0)
    for j in range(15, -1, -1):
        cs = css[j] + _splat(carries[j])
        hit = cs >= target_s
        if exclude_zero and j == 0:
            hit = hit & (lane_iota != 15)
        anyhit = _scalar(plsc.all_reduce_population_count(hit)) > 0
        istar = plsc.all_reduce_ffs(hit)
        cs_at = _take0(cs, istar)
        rc_at = _take0(rcs[j], istar)
        newly = anyhit & jnp.logical_not(found)
        B = jnp.where(newly, 16 * j + 15 - _scalar(istar), B)
        above = jnp.where(newly, cs_at - rc_at, above)
        found = found | anyhit
    return found, B, target - above


def _sc_body(x_hbm, o_hbm, row_v, hist_v, cb1_v, cb2_v):
    wid = lax.axis_index("s") * 2 + lax.axis_index("c")
    lane_iota = lax.iota(jnp.int32, 16)
    ones16 = jnp.ones((16,), jnp.int32)
    zeros16 = jnp.zeros((16,), jnp.int32)

    def reset_hist():
        for j in range(16):
            hist_v[pl.ds(16 * j, 16)] = zeros16

    def do_row(rr, _):
        orow = wid * _ROWS_PER_W + rr
        row = _SPLIT + orow
        pltpu.sync_copy(x_hbm.at[row], row_v)

        reset_hist()

        def passA(i, _):
            v = row_v[pl.ds(i * 16, 16)]
            bits = plsc.bitcast(jnp.maximum(v, 0.0), jnp.int32)
            e = lax.shift_right_logical(bits, 23)
            plsc.addupdate_scatter(hist_v, [e], ones16)
            return 0
        lax.fori_loop(0, _CHUNKS, passA, 0, unroll=8)

        found1, B1, r1 = _scan_hist(hist_v, jnp.int32(_K), True, lane_iota)
        B1 = jnp.where(found1, B1, -1)

        reset_hist()
        B1s = _splat(B1)

        def passB(i, w):
            v = row_v[pl.ds(i * 16, 16)]
            bits = plsc.bitcast(jnp.maximum(v, 0.0), jnp.int32)
            e = lax.shift_right_logical(bits, 23)
            match = e == B1s
            b2 = lax.shift_right_logical(bits, 15) & 0xFF
            plsc.addupdate_scatter(hist_v, [b2], ones16,
                                   mask=match)
            plsc.store_compressed(cb1_v.at[pl.ds(w, 16)], bits, mask=match)
            return w + _scalar(plsc.all_reduce_population_count(match))
        m1 = lax.fori_loop(0, _CHUNKS, passB, jnp.int32(0), unroll=8)

        found2, B2, r2 = _scan_hist(hist_v, r1, False, lane_iota)

        reset_hist()
        B2s = _splat(B2)
        m1s = _splat(m1)
        trip1 = lax.shift_right_logical(m1 + 15, 4)

        def passC(i, w):
            bits = cb1_v[pl.ds(i * 16, 16)]
            valid = (i * 16 + lane_iota) < m1s
            b2 = lax.shift_right_logical(bits, 15) & 0xFF
            match = valid & (b2 == B2s)
            b3 = lax.shift_right_logical(bits, 7) & 0xFF
            plsc.addupdate_scatter(hist_v, [b3], ones16,
                                   mask=match)
            plsc.store_compressed(cb2_v.at[pl.ds(w, 16)], bits, mask=match)
            return w + _scalar(plsc.all_reduce_population_count(match))
        m2 = lax.fori_loop(0, trip1, passC, jnp.int32(0))

        found3, B3, r3 = _scan_hist(hist_v, r2, False, lane_iota)

        reset_hist()
        B3s = _splat(B3)
        m2s = _splat(m2)
        trip2 = lax.shift_right_logical(m2 + 15, 4)

        def passD(i, _):
            bits = cb2_v[pl.ds(i * 16, 16)]
            valid = (i * 16 + lane_iota) < m2s
            b3 = lax.shift_right_logical(bits, 7) & 0xFF
            match = valid & (b3 == B3s)
            b4 = bits & 0x7F
            plsc.addupdate_scatter(hist_v, [b4], ones16,
                                   mask=match)
            return 0
        lax.fori_loop(0, trip2, passD, 0)

        found4, B4, _r4 = _scan_hist(hist_v, r3, False, lane_iota)

        t_bits = (B1 << 23) | (B2 << 15) | (B3 << 7) | B4
        t_bits = jnp.where(found1, t_bits, jnp.int32(1))
        tf = plsc.bitcast(_splat(t_bits), jnp.float32)

        def passE(i, _):
            v = row_v[pl.ds(i * 16, 16)]
            row_v[pl.ds(i * 16, 16)] = jnp.where(v >= tf, v, 0.0)
            return 0
        lax.fori_loop(0, _CHUNKS, passE, 0, unroll=8)

        pltpu.sync_copy(row_v, o_hbm.at[orow])
        return 0

    lax.fori_loop(0, _ROWS_PER_W, do_row, 0)


def _tc_block(x_ref, o_ref):
    x = x_ref[...]
    rows = x.shape[0]

    def step(bit, t_bits, cur_cnt):
        cand = t_bits | (jnp.int32(1) << bit)
        cand_f = jax.lax.bitcast_convert_type(cand, jnp.float32)
        cnt = jnp.sum((x >= cand_f).astype(jnp.int32), axis=1, keepdims=True)
        take = cnt >= _K
        return (jnp.where(take, cand, t_bits), jnp.where(take, cnt, cur_cnt))

    def head(i, carry):
        t_bits, cur_cnt = carry
        return step(30 - i, t_bits, cur_cnt)

    t0 = jnp.zeros((rows, 1), jnp.int32)
    c0 = jnp.full((rows, 1), _N, jnp.int32)
    t_bits, cur_cnt = jax.lax.fori_loop(0, 18, head, (t0, c0), unroll=True)

    def cond(carry):
        bit, _, cur_cnt = carry
        return (bit >= 0) & jnp.any(cur_cnt != _K)

    def body(carry):
        bit, t_bits, cur_cnt = carry
        t_bits, cur_cnt = step(bit, t_bits, cur_cnt)
        return (bit - 1, t_bits, cur_cnt)

    _, t_bits, _ = jax.lax.while_loop(cond, body, (jnp.int32(12), t_bits, cur_cnt))
    t_f = jax.lax.bitcast_convert_type(t_bits, jnp.float32)
    keep = (x >= t_f) & (x > 0.0)
    o_ref[...] = jnp.where(keep, x, 0.0)


@jax.jit
def kernel(x):
    sc_out = pl.kernel(
        _sc_body,
        out_type=jax.ShapeDtypeStruct((_B - _SPLIT, _N), jnp.float32),
        mesh=plsc.VectorSubcoreMesh(core_axis_name="c", subcore_axis_name="s"),
        compiler_params=pltpu.CompilerParams(needs_layout_passes=False),
        scratch_types=[
            pltpu.VMEM((_N,), jnp.float32),
            pltpu.VMEM((256,), jnp.int32),
            pltpu.VMEM((_N + 16,), jnp.int32),
            pltpu.VMEM((_N + 16,), jnp.int32),
        ],
    )(x)
    tc_full = pl.pallas_call(
        _tc_block,
        grid=(_SPLIT // _TC_ROWS_PER_BLOCK,),
        in_specs=[pl.BlockSpec((_TC_ROWS_PER_BLOCK, _N), lambda i: (i, 0))],
        out_specs=pl.BlockSpec((_TC_ROWS_PER_BLOCK, _N), lambda i: (i, 0)),
        out_shape=jax.ShapeDtypeStruct((_B, _N), x.dtype),
    )(x)
    return jax.lax.dynamic_update_slice(tc_full, sc_out, (_SPLIT, 0))
